# baseline (speedup 1.0000x reference)
import functools

import jax
import jax.numpy as jnp
from jax import lax
from jax.experimental import pallas as pl
from jax.experimental.pallas import tpu as pltpu
from jax.experimental.pallas import tpu_sc as plsc

_NC, _NS = 2, 16
_NW = _NC * _NS


def _sc_sum_body(atoms_hbm, bonds_hbm, out_hbm, at_v, bt_v, st_v,
                 *, S, A, F, D):
    wid = lax.axis_index("s") * _NC + lax.axis_index("c")
    spw = S // _NW
    zero16 = jnp.zeros((16,), jnp.float32)
    for j in range(F // 16):
        at_v[pl.ds(A * F + j * 16, 16)] = zero16
    lane = lax.iota(jnp.int32, 16)

    def per_sample(si, carry):
        s = wid * spw + si
        pltpu.sync_copy(atoms_hbm.at[s], at_v.at[pl.ds(0, A * F)])
        pltpu.sync_copy(bonds_hbm.at[s], bt_v)

        def per_chunk(ch, carry2):
            a0 = ch * 16
            base_self = (a0 + lane) * F
            bases = []
            for d in range(D):
                bi = bt_v[d, pl.ds(a0, 16)]
                bases.append(jnp.where(bi < 0, A, bi) * F)
            for f in range(F):
                fv = jnp.full((16,), f, jnp.int32)
                acc0 = plsc.load_gather(at_v, [base_self + fv])
                acc1 = plsc.load_gather(at_v, [bases[0] + fv])
                for d in range(1, D):
                    g = plsc.load_gather(at_v, [bases[d] + fv])
                    if d % 2:
                        acc0 = acc0 + g
                    else:
                        acc1 = acc1 + g
                st_v[f, pl.ds(a0, 16)] = acc0 + acc1
            return carry2

        lax.fori_loop(0, A // 16, per_chunk, 0)
        pltpu.sync_copy(st_v, out_hbm.at[s])
        return carry

    lax.fori_loop(0, spw, per_sample, 0)


def _dense_body(st_ref, bonds_ref, wp_ref, b_ref, out_ref, *, A, D, C):
    st = st_ref[0]
    bb = bonds_ref[0]
    full = lax.dot_general(st, wp_ref[...], (((0,), (0,)), ((), ())),
                           preferred_element_type=jnp.float32)
    full = full + b_ref[...]
    deg = jnp.sum((bb != -1).astype(jnp.int32), axis=1, keepdims=True)
    acc = jnp.zeros((A, C), jnp.float32)
    for d in range(D):
        acc = acc + jnp.where(deg == d, full[:, d * C:(d + 1) * C], 0.0)
    out_ref[0] = acc


def kernel(atoms, bonds, Ws, bs):
    S, A, F = atoms.shape
    D, _, C = Ws.shape
    wp = jnp.transpose(Ws, (1, 0, 2)).reshape(F, D * C)
    br = bs.reshape(1, D * C)
    bonds32 = bonds.astype(jnp.int32)
    bonds_t = jnp.transpose(bonds32, (0, 2, 1))

    mesh = plsc.VectorSubcoreMesh(core_axis_name="c", subcore_axis_name="s",
                                  num_cores=_NC, num_subcores=_NS)
    summed_t = pl.kernel(
        functools.partial(_sc_sum_body, S=S, A=A, F=F, D=D),
        out_type=jax.ShapeDtypeStruct((S, F, A), jnp.float32),
        mesh=mesh,
        compiler_params=pltpu.CompilerParams(needs_layout_passes=False),
        scratch_types=[
            pltpu.VMEM(((A + 1) * F,), jnp.float32),
            pltpu.VMEM((D, A), jnp.int32),
            pltpu.VMEM((F, A), jnp.float32),
        ],
    )(atoms.reshape(S, A * F), bonds_t)

    return pl.pallas_call(
        functools.partial(_dense_body, A=A, D=D, C=C),
        grid=(S,),
        in_specs=[
            pl.BlockSpec((1, F, A), lambda s: (s, 0, 0)),
            pl.BlockSpec((1, A, D), lambda s: (s, 0, 0)),
            pl.BlockSpec((F, D * C), lambda s: (0, 0)),
            pl.BlockSpec((1, D * C), lambda s: (0, 0)),
        ],
        out_specs=pl.BlockSpec((1, A, C), lambda s: (s, 0, 0)),
        out_shape=jax.ShapeDtypeStruct((S, A, C), jnp.float32),
    )(summed_t, bonds32, wp, br)


# speedup vs baseline: 1.1459x; 1.1459x over previous
import functools

import jax
import jax.numpy as jnp
from jax import lax
from jax.experimental import pallas as pl
from jax.experimental.pallas import tpu as pltpu
from jax.experimental.pallas import tpu_sc as plsc

_NC, _NS = 2, 16
_NW = _NC * _NS


def _sc_sum_body(atoms_hbm, bonds_hbm, out_hbm, at_v, bt_v, st_v,
                 *, S, A, F, D):
    wid = lax.axis_index("s") * _NC + lax.axis_index("c")
    spw = S // _NW
    zero16 = jnp.zeros((16,), jnp.float32)
    for j in range(F // 16):
        at_v[pl.ds(A * F + j * 16, 16)] = zero16
    lane = lax.iota(jnp.int32, 16)

    def per_sample(si, carry):
        s = wid * spw + si
        pltpu.sync_copy(atoms_hbm.at[s], at_v.at[pl.ds(0, A * F)])
        pltpu.sync_copy(bonds_hbm.at[s], bt_v)

        def per_chunk(ch, carry2):
            a0 = ch * 16
            base_self = (a0 + lane) * F
            bases = []
            for d in range(D):
                bi = bt_v[d, pl.ds(a0, 16)]
                bases.append(jnp.where(bi < 0, A, bi) * F)

            @plsc.parallel_loop(0, F, unroll=4)
            def per_f(f):
                fv = jnp.full((16,), f, jnp.int32)
                acc0 = plsc.load_gather(at_v, [base_self + fv])
                acc1 = plsc.load_gather(at_v, [bases[0] + fv])
                for d in range(1, D):
                    g = plsc.load_gather(at_v, [bases[d] + fv])
                    if d % 2:
                        acc0 = acc0 + g
                    else:
                        acc1 = acc1 + g
                st_v[f, pl.ds(a0, 16)] = acc0 + acc1

            return carry2

        lax.fori_loop(0, A // 16, per_chunk, 0)
        pltpu.sync_copy(st_v, out_hbm.at[s])
        return carry

    lax.fori_loop(0, spw, per_sample, 0)


def _dense_body(st_ref, bonds_ref, wp_ref, b_ref, out_ref, *, A, D, C):
    st = st_ref[0]
    bb = bonds_ref[0]
    full = lax.dot_general(st, wp_ref[...], (((0,), (0,)), ((), ())),
                           preferred_element_type=jnp.float32)
    full = full + b_ref[...]
    deg = jnp.sum((bb != -1).astype(jnp.int32), axis=1, keepdims=True)
    acc = jnp.zeros((A, C), jnp.float32)
    for d in range(D):
        acc = acc + jnp.where(deg == d, full[:, d * C:(d + 1) * C], 0.0)
    out_ref[0] = acc


def kernel(atoms, bonds, Ws, bs):
    S, A, F = atoms.shape
    D, _, C = Ws.shape
    wp = jnp.transpose(Ws, (1, 0, 2)).reshape(F, D * C)
    br = bs.reshape(1, D * C)
    bonds32 = bonds.astype(jnp.int32)
    bonds_t = jnp.transpose(bonds32, (0, 2, 1))

    mesh = plsc.VectorSubcoreMesh(core_axis_name="c", subcore_axis_name="s",
                                  num_cores=_NC, num_subcores=_NS)
    summed_t = pl.kernel(
        functools.partial(_sc_sum_body, S=S, A=A, F=F, D=D),
        out_type=jax.ShapeDtypeStruct((S, F, A), jnp.float32),
        mesh=mesh,
        compiler_params=pltpu.CompilerParams(needs_layout_passes=False),
        scratch_types=[
            pltpu.VMEM(((A + 1) * F,), jnp.float32),
            pltpu.VMEM((D, A), jnp.int32),
            pltpu.VMEM((F, A), jnp.float32),
        ],
    )(atoms.reshape(S, A * F), bonds_t)

    return pl.pallas_call(
        functools.partial(_dense_body, A=A, D=D, C=C),
        grid=(S,),
        in_specs=[
            pl.BlockSpec((1, F, A), lambda s: (s, 0, 0)),
            pl.BlockSpec((1, A, D), lambda s: (s, 0, 0)),
            pl.BlockSpec((F, D * C), lambda s: (0, 0)),
            pl.BlockSpec((1, D * C), lambda s: (0, 0)),
        ],
        out_specs=pl.BlockSpec((1, A, C), lambda s: (s, 0, 0)),
        out_shape=jax.ShapeDtypeStruct((S, A, C), jnp.float32),
    )(summed_t, bonds32, wp, br)


# speedup vs baseline: 3.6913x; 3.2213x over previous
import functools

import jax
import jax.numpy as jnp
from jax import lax
from jax.experimental import pallas as pl
from jax.experimental.pallas import tpu as pltpu
from jax.experimental.pallas import tpu_sc as plsc

_NC, _NS = 2, 16
_NW = _NC * _NS


def _sc_sum_body(atoms_hbm, bonds_hbm, out_hbm, at_v, bd_v, st_v,
                 *, S, A, F, D):
    wid = lax.axis_index("s") * _NC + lax.axis_index("c")
    spw = S // _NW
    nf = F // 16
    zero16 = jnp.zeros((16,), jnp.float32)
    for j in range(nf):
        at_v[pl.ds(A * F + j * 16, 16)] = zero16

    def per_sample(si, carry):
        s = wid * spw + si
        pltpu.sync_copy(atoms_hbm.at[s], at_v.at[pl.ds(0, A * F)])
        pltpu.sync_copy(bonds_hbm.at[s], bd_v)

        @plsc.parallel_loop(0, A)
        def per_atom(a):
            brow = bd_v[a, pl.ds(0, D)]
            bases = jnp.where(brow < 0, A, brow) * F
            accs = [at_v[pl.ds(a * F + 16 * j, 16)] for j in range(nf)]
            for d in range(D):
                base = bases[d]
                for j in range(nf):
                    accs[j] = accs[j] + at_v[pl.ds(base + 16 * j, 16)]
            for j in range(nf):
                st_v[a, pl.ds(16 * j, 16)] = accs[j]

        pltpu.sync_copy(st_v, out_hbm.at[s])
        return carry

    lax.fori_loop(0, spw, per_sample, 0)


def _dense_body(sum_ref, bonds_ref, wp_ref, b_ref, out_ref, *, A, D, C):
    summed = sum_ref[0]
    bb = bonds_ref[0]
    full = lax.dot_general(summed, wp_ref[...], (((1,), (0,)), ((), ())),
                           preferred_element_type=jnp.float32)
    full = full + b_ref[...]
    deg = jnp.sum((bb != -1).astype(jnp.int32), axis=1, keepdims=True)
    acc = jnp.zeros((A, C), jnp.float32)
    for d in range(D):
        acc = acc + jnp.where(deg == d, full[:, d * C:(d + 1) * C], 0.0)
    out_ref[0] = acc


def kernel(atoms, bonds, Ws, bs):
    S, A, F = atoms.shape
    D, _, C = Ws.shape
    wp = jnp.transpose(Ws, (1, 0, 2)).reshape(F, D * C)
    br = bs.reshape(1, D * C)
    bonds32 = bonds.astype(jnp.int32)

    mesh = plsc.VectorSubcoreMesh(core_axis_name="c", subcore_axis_name="s",
                                  num_cores=_NC, num_subcores=_NS)
    summed = pl.kernel(
        functools.partial(_sc_sum_body, S=S, A=A, F=F, D=D),
        out_type=jax.ShapeDtypeStruct((S, A, F), jnp.float32),
        mesh=mesh,
        compiler_params=pltpu.CompilerParams(needs_layout_passes=False),
        scratch_types=[
            pltpu.VMEM(((A + 1) * F,), jnp.float32),
            pltpu.VMEM((A, D), jnp.int32),
            pltpu.VMEM((A, F), jnp.float32),
        ],
    )(atoms.reshape(S, A * F), bonds32)

    return pl.pallas_call(
        functools.partial(_dense_body, A=A, D=D, C=C),
        grid=(S,),
        in_specs=[
            pl.BlockSpec((1, A, F), lambda s: (s, 0, 0)),
            pl.BlockSpec((1, A, D), lambda s: (s, 0, 0)),
            pl.BlockSpec((F, D * C), lambda s: (0, 0)),
            pl.BlockSpec((1, D * C), lambda s: (0, 0)),
        ],
        out_specs=pl.BlockSpec((1, A, C), lambda s: (s, 0, 0)),
        out_shape=jax.ShapeDtypeStruct((S, A, C), jnp.float32),
    )(summed, bonds32, wp, br)


# speedup vs baseline: 4.8745x; 1.3205x over previous
import functools

import jax
import jax.numpy as jnp
from jax import lax
from jax.experimental import pallas as pl
from jax.experimental.pallas import tpu as pltpu
from jax.experimental.pallas import tpu_sc as plsc

_NC, _NS = 2, 16
_NW = _NC * _NS


def _sc_sum_body(atoms_hbm, bonds_hbm, out_hbm, at_v, bd_v, st_v,
                 *, S, A, F, D):
    wid = lax.axis_index("s") * _NC + lax.axis_index("c")
    spw = S // _NW
    nf = F // 16
    zero16 = jnp.zeros((16,), jnp.float32)
    for j in range(nf):
        at_v[A, pl.ds(16 * j, 16)] = zero16

    def per_sample(si, carry):
        s = wid * spw + si
        pltpu.sync_copy(atoms_hbm.at[s], at_v.at[pl.ds(0, A)])
        pltpu.sync_copy(bonds_hbm.at[s], bd_v)

        @plsc.parallel_loop(0, A)
        def per_atom(a):
            brow = bd_v[a, pl.ds(0, D)]
            rows = jnp.where(brow < 0, A, brow)
            accs = [at_v[a, pl.ds(16 * j, 16)] for j in range(nf)]
            for d in range(D):
                r = rows[d]
                for j in range(nf):
                    accs[j] = accs[j] + at_v[r, pl.ds(16 * j, 16)]
            for j in range(nf):
                st_v[a, pl.ds(16 * j, 16)] = accs[j]

        pltpu.sync_copy(st_v, out_hbm.at[s])
        return carry

    lax.fori_loop(0, spw, per_sample, 0)


def _dense_body(sum_ref, bonds_ref, wp_ref, b_ref, out_ref, *, R, D, C):
    summed = sum_ref[...]
    bb = bonds_ref[...]
    full = lax.dot_general(summed, wp_ref[...], (((1,), (0,)), ((), ())),
                           preferred_element_type=jnp.float32)
    full = full + b_ref[...]
    deg = jnp.sum((bb != -1).astype(jnp.int32), axis=1, keepdims=True)
    acc = jnp.zeros((R, C), jnp.float32)
    for d in range(D):
        acc = acc + jnp.where(deg == d, full[:, d * C:(d + 1) * C], 0.0)
    out_ref[...] = acc


def kernel(atoms, bonds, Ws, bs):
    S, A, F = atoms.shape
    D, _, C = Ws.shape
    wp = jnp.transpose(Ws, (1, 0, 2)).reshape(F, D * C)
    br = bs.reshape(1, D * C)
    bonds32 = bonds.astype(jnp.int32)

    mesh = plsc.VectorSubcoreMesh(core_axis_name="c", subcore_axis_name="s",
                                  num_cores=_NC, num_subcores=_NS)
    summed = pl.kernel(
        functools.partial(_sc_sum_body, S=S, A=A, F=F, D=D),
        out_type=jax.ShapeDtypeStruct((S, A, F), jnp.float32),
        mesh=mesh,
        compiler_params=pltpu.CompilerParams(needs_layout_passes=False),
        scratch_types=[
            pltpu.VMEM((A + 1, F), jnp.float32),
            pltpu.VMEM((A, D), jnp.int32),
            pltpu.VMEM((A, F), jnp.float32),
        ],
    )(atoms, bonds32)

    R = 1024
    N = S * A
    out = pl.pallas_call(
        functools.partial(_dense_body, R=R, D=D, C=C),
        grid=(N // R,),
        in_specs=[
            pl.BlockSpec((R, F), lambda r: (r, 0)),
            pl.BlockSpec((R, D), lambda r: (r, 0)),
            pl.BlockSpec((F, D * C), lambda r: (0, 0)),
            pl.BlockSpec((1, D * C), lambda r: (0, 0)),
        ],
        out_specs=pl.BlockSpec((R, C), lambda r: (r, 0)),
        out_shape=jax.ShapeDtypeStruct((N, C), jnp.float32),
    )(summed.reshape(N, F), bonds32.reshape(N, D), wp, br)
    return out.reshape(S, A, C)


# speedup vs baseline: 5.8650x; 1.2032x over previous
import functools

import jax
import jax.numpy as jnp
from jax import lax
from jax.experimental import pallas as pl
from jax.experimental.pallas import tpu as pltpu
from jax.experimental.pallas import tpu_sc as plsc

_NC, _NS = 2, 16
_NW = _NC * _NS
_NSLICE = 2
_R = 1024


def _sc_sum_body(atoms_hbm, bonds_hbm, out_hbm,
                 at0, at1, bd0, bd1, st_v, sin0, sin1,
                 *, base, spw, A, F, D):
    wid = lax.axis_index("s") * _NC + lax.axis_index("c")
    s0 = base + wid * spw
    nf = F // 16
    ats = (at0, at1)
    bds = (bd0, bd1)
    sins = (sin0, sin1)
    zero16 = jnp.zeros((16,), jnp.float32)
    for j in range(nf):
        at0[A, pl.ds(16 * j, 16)] = zero16
        at1[A, pl.ds(16 * j, 16)] = zero16

    def issue_in(si, p):
        pltpu.async_copy(atoms_hbm.at[s0 + si], ats[p].at[pl.ds(0, A)], sins[p])
        pltpu.async_copy(bonds_hbm.at[s0 + si], bds[p], sins[p])

    def wait_in(si, p):
        pltpu.make_async_copy(atoms_hbm.at[s0 + si],
                              ats[p].at[pl.ds(0, A)], sins[p]).wait()
        pltpu.make_async_copy(bonds_hbm.at[s0 + si], bds[p], sins[p]).wait()

    issue_in(0, 0)

    def pair_body(i, carry):
        for par in range(2):
            si = 2 * i + par
            wait_in(si, par)

            @pl.when(si + 1 < spw)
            def _():
                issue_in(si + 1, 1 - par)

            at_v = ats[par]
            bd_v = bds[par]

            @plsc.parallel_loop(0, A)
            def per_atom(a):
                brow = bd_v[a, pl.ds(0, D)]
                rows = jnp.where(brow < 0, A, brow)
                accs = [at_v[a, pl.ds(16 * j, 16)] for j in range(nf)]
                for d in range(D):
                    r = rows[d]
                    for j in range(nf):
                        accs[j] = accs[j] + at_v[r, pl.ds(16 * j, 16)]
                for j in range(nf):
                    st_v[a, pl.ds(16 * j, 16)] = accs[j]

            pltpu.sync_copy(st_v, out_hbm.at[wid * spw + si])
        return carry

    lax.fori_loop(0, spw // 2, pair_body, 0)


def _dense_body(sum_ref, bonds_ref, wp_ref, b_ref, out_ref, *, R, D, C):
    summed = sum_ref[...]
    bb = bonds_ref[...]
    full = lax.dot_general(summed, wp_ref[...], (((1,), (0,)), ((), ())),
                           preferred_element_type=jnp.float32)
    full = full + b_ref[...]
    deg = jnp.sum((bb != -1).astype(jnp.int32), axis=1, keepdims=True)
    acc = jnp.zeros((R, C), jnp.float32)
    for d in range(D):
        acc = acc + jnp.where(deg == d, full[:, d * C:(d + 1) * C], 0.0)
    out_ref[...] = acc


def kernel(atoms, bonds, Ws, bs):
    S, A, F = atoms.shape
    D, _, C = Ws.shape
    wp = jnp.transpose(Ws, (1, 0, 2)).reshape(F, D * C)
    br = bs.reshape(1, D * C)
    bonds32 = bonds.astype(jnp.int32)
    bonds_rows = bonds32.reshape(S * A, D)

    mesh = plsc.VectorSubcoreMesh(core_axis_name="c", subcore_axis_name="s",
                                  num_cores=_NC, num_subcores=_NS)
    ssl = S // _NSLICE
    spw = ssl // _NW
    outs = []
    for k in range(_NSLICE):
        summed_k = pl.kernel(
            functools.partial(_sc_sum_body, base=k * ssl, spw=spw,
                              A=A, F=F, D=D),
            out_type=jax.ShapeDtypeStruct((ssl, A, F), jnp.float32),
            mesh=mesh,
            compiler_params=pltpu.CompilerParams(needs_layout_passes=False),
            scratch_types=[
                pltpu.VMEM((A + 1, F), jnp.float32),
                pltpu.VMEM((A + 1, F), jnp.float32),
                pltpu.VMEM((A, D), jnp.int32),
                pltpu.VMEM((A, D), jnp.int32),
                pltpu.VMEM((A, F), jnp.float32),
                pltpu.SemaphoreType.DMA,
                pltpu.SemaphoreType.DMA,
            ],
        )(atoms, bonds32)

        rbase = k * ssl * A // _R
        out_k = pl.pallas_call(
            functools.partial(_dense_body, R=_R, D=D, C=C),
            grid=(ssl * A // _R,),
            in_specs=[
                pl.BlockSpec((_R, F), lambda r: (r, 0)),
                pl.BlockSpec((_R, D), lambda r, rb=rbase: (rb + r, 0)),
                pl.BlockSpec((F, D * C), lambda r: (0, 0)),
                pl.BlockSpec((1, D * C), lambda r: (0, 0)),
            ],
            out_specs=pl.BlockSpec((_R, C), lambda r: (r, 0)),
            out_shape=jax.ShapeDtypeStruct((ssl * A, C), jnp.float32),
        )(summed_k.reshape(ssl * A, F), bonds_rows, wp, br)
        outs.append(out_k.reshape(ssl, A, C))
    return jnp.concatenate(outs, axis=0)


# speedup vs baseline: 6.5192x; 1.1116x over previous
import functools

import jax
import jax.numpy as jnp
from jax import lax
from jax.experimental import pallas as pl
from jax.experimental.pallas import tpu as pltpu
from jax.experimental.pallas import tpu_sc as plsc

_NC, _NS = 2, 16
_NW = _NC * _NS
_NSLICE = 2
_R = 1024


def _sc_sum_body(atoms_hbm, bonds_hbm, out_hbm,
                 at0, at1, bd0, bd1, st0, st1, sin0, sin1, sout0, sout1,
                 *, base, spw, A, F, D):
    wid = lax.axis_index("s") * _NC + lax.axis_index("c")
    s0 = base + wid * spw
    o0 = wid * spw
    nf = F // 16
    ats = (at0, at1)
    bds = (bd0, bd1)
    sts = (st0, st1)
    sins = (sin0, sin1)
    souts = (sout0, sout1)
    zero16 = jnp.zeros((16,), jnp.float32)
    for j in range(nf):
        at0[A, pl.ds(16 * j, 16)] = zero16
        at1[A, pl.ds(16 * j, 16)] = zero16

    def issue_in(si, p):
        pltpu.async_copy(atoms_hbm.at[s0 + si], ats[p].at[pl.ds(0, A)], sins[p])
        pltpu.async_copy(bonds_hbm.at[s0 + si], bds[p], sins[p])

    def wait_in(si, p):
        pltpu.make_async_copy(atoms_hbm.at[s0 + si],
                              ats[p].at[pl.ds(0, A)], sins[p]).wait()
        pltpu.make_async_copy(bonds_hbm.at[s0 + si], bds[p], sins[p]).wait()

    def wait_out(si, p):
        pltpu.make_async_copy(sts[p], out_hbm.at[o0 + si], souts[p]).wait()

    issue_in(0, 0)

    def pair_body(i, carry):
        for par in range(2):
            si = 2 * i + par
            wait_in(si, par)

            @pl.when(si + 1 < spw)
            def _():
                issue_in(si + 1, 1 - par)

            @pl.when(si >= 2)
            def _():
                wait_out(si - 2, par)

            at_v = ats[par]
            bd_v = bds[par]
            st_v = sts[par]

            @plsc.parallel_loop(0, A, unroll=2)
            def per_atom(a):
                brow = bd_v[a, pl.ds(0, D)]
                rows = jnp.where(brow < 0, A, brow)
                accs = [at_v[a, pl.ds(16 * j, 16)] for j in range(nf)]
                for d in range(D):
                    r = rows[d]
                    for j in range(nf):
                        accs[j] = accs[j] + at_v[r, pl.ds(16 * j, 16)]
                for j in range(nf):
                    st_v[a, pl.ds(16 * j, 16)] = accs[j]

            pltpu.async_copy(st_v, out_hbm.at[o0 + si], souts[par])
        return carry

    lax.fori_loop(0, spw // 2, pair_body, 0)
    wait_out(spw - 2, 0)
    wait_out(spw - 1, 1)


def _dense_body(sum_ref, bonds_ref, wp_ref, b_ref, out_ref, *, R, D, C):
    summed = sum_ref[...]
    bb = bonds_ref[...]
    full = lax.dot_general(summed, wp_ref[...], (((1,), (0,)), ((), ())),
                           preferred_element_type=jnp.float32)
    full = full + b_ref[...]
    deg = jnp.sum((bb != -1).astype(jnp.int32), axis=1, keepdims=True)
    acc = jnp.zeros((R, C), jnp.float32)
    for d in range(D):
        acc = acc + jnp.where(deg == d, full[:, d * C:(d + 1) * C], 0.0)
    out_ref[...] = acc


def kernel(atoms, bonds, Ws, bs):
    S, A, F = atoms.shape
    D, _, C = Ws.shape
    wp = jnp.transpose(Ws, (1, 0, 2)).reshape(F, D * C)
    br = bs.reshape(1, D * C)
    bonds32 = bonds.astype(jnp.int32)
    bonds_rows = bonds32.reshape(S * A, D)

    mesh = plsc.VectorSubcoreMesh(core_axis_name="c", subcore_axis_name="s",
                                  num_cores=_NC, num_subcores=_NS)
    ssl = S // _NSLICE
    spw = ssl // _NW
    outs = []
    for k in range(_NSLICE):
        summed_k = pl.kernel(
            functools.partial(_sc_sum_body, base=k * ssl, spw=spw,
                              A=A, F=F, D=D),
            out_type=jax.ShapeDtypeStruct((ssl, A, F), jnp.float32),
            mesh=mesh,
            compiler_params=pltpu.CompilerParams(needs_layout_passes=False),
            scratch_types=[
                pltpu.VMEM((A + 1, F), jnp.float32),
                pltpu.VMEM((A + 1, F), jnp.float32),
                pltpu.VMEM((A, D), jnp.int32),
                pltpu.VMEM((A, D), jnp.int32),
                pltpu.VMEM((A, F), jnp.float32),
                pltpu.VMEM((A, F), jnp.float32),
                pltpu.SemaphoreType.DMA,
                pltpu.SemaphoreType.DMA,
                pltpu.SemaphoreType.DMA,
                pltpu.SemaphoreType.DMA,
            ],
        )(atoms, bonds32)

        rbase = k * ssl * A // _R
        out_k = pl.pallas_call(
            functools.partial(_dense_body, R=_R, D=D, C=C),
            grid=(ssl * A // _R,),
            in_specs=[
                pl.BlockSpec((_R, F), lambda r: (r, 0)),
                pl.BlockSpec((_R, D), lambda r, rb=rbase: (rb + r, 0)),
                pl.BlockSpec((F, D * C), lambda r: (0, 0)),
                pl.BlockSpec((1, D * C), lambda r: (0, 0)),
            ],
            out_specs=pl.BlockSpec((_R, C), lambda r: (r, 0)),
            out_shape=jax.ShapeDtypeStruct((ssl * A, C), jnp.float32),
        )(summed_k.reshape(ssl * A, F), bonds_rows, wp, br)
        outs.append(out_k.reshape(ssl, A, C))
    return jnp.concatenate(outs, axis=0)
